# baseline (device time: 36422 ns/iter reference)
import jax
import jax.numpy as jnp
from jax import lax
from jax.experimental import pallas as pl
from jax.experimental.pallas import tpu as pltpu

N_DEV = 8
B, SQ, SKV, D = 2, 256, 512, 768
HQ_PER = 8
DH = 64
DM = HQ_PER * DH
SCALE = 0.125
ROWS = B * SQ
CH = ROWS // N_DEV

_MESH = pl.DeviceIdType.MESH
_ALLREDUCE = True


def kernel(x, Wq, Wo, K_ext, V_ext):
    my_sm = lax.axis_index("i")
    h0 = my_sm * HQ_PER
    Kmy = lax.dynamic_slice_in_dim(K_ext, h0, HQ_PER, axis=2)
    Vmy = lax.dynamic_slice_in_dim(V_ext, h0, HQ_PER, axis=2)
    Kmy = Kmy.astype(jnp.bfloat16).reshape(B, SKV, DM)
    Vmy = Vmy.astype(jnp.bfloat16).reshape(B, SKV, DM)

    def body(x_ref, wq_ref, wo_ref, k_ref, v_ref, out_ref,
             attn_ref, acc_ref, sbuf, cbuf, send_sems, recv_sems):
        my = lax.axis_index("i")
        r = my & 3
        myx = (r ^ (r >> 1)) & 1
        myy = r >> 1
        myz = my >> 2
        p_x = my ^ 1
        p_y = my ^ 3
        p_z = my ^ 4

        with jax.named_scope("qgemm"):
            x2 = x_ref[...].reshape(ROWS, D).astype(jnp.bfloat16)
            q_all = lax.dot_general(
                x2, wq_ref[...].astype(jnp.bfloat16),
                (((1,), (0,)), ((), ())),
                preferred_element_type=jnp.float32).astype(jnp.bfloat16)

        pending = []

        def wait_recv(j):
            rdma = pltpu.make_async_remote_copy(
                src_ref=sbuf.at[0], dst_ref=cbuf.at[j],
                send_sem=send_sems.at[j], recv_sem=recv_sems.at[j],
                device_id=(my,), device_id_type=_MESH)
            rdma.wait_recv()

        for b in range(B):
            with jax.named_scope(f"attn{b}"):
                kb = k_ref[b]
                vb = v_ref[b]
                for h in range(HQ_PER):
                    q_bh = q_all[b * SQ:(b + 1) * SQ, h * DH:(h + 1) * DH]
                    k_bh = kb[:, h * DH:(h + 1) * DH]
                    v_bh = vb[:, h * DH:(h + 1) * DH]
                    s = lax.dot_general(
                        q_bh, k_bh, (((1,), (1,)), ((), ())),
                        preferred_element_type=jnp.float32) * SCALE
                    m = jnp.max(s, axis=1, keepdims=True)
                    p = jnp.exp(s - m)
                    l = jnp.sum(p, axis=1, keepdims=True)
                    o = lax.dot_general(
                        p.astype(jnp.bfloat16), v_bh, (((1,), (0,)), ((), ())),
                        preferred_element_type=jnp.float32)
                    attn_ref[pl.ds(b * SQ, SQ), pl.ds(h * DH, DH)] = (
                        o / l).astype(jnp.bfloat16)

            with jax.named_scope(f"ogemm{b}"):
                acc_ref[b * SQ:(b + 1) * SQ, :] = lax.dot_general(
                    attn_ref[b * SQ:(b + 1) * SQ, :],
                    wo_ref[...].astype(jnp.bfloat16),
                    (((1,), (0,)), ((), ())),
                    preferred_element_type=jnp.float32)

            if not _ALLREDUCE:
                out_ref[b] = acc_ref[b * SQ:(b + 1) * SQ, :]
                continue

            with jax.named_scope(f"rs_send{b}"):
                for d in range(1, N_DEV):
                    p = lax.rem(my + d, N_DEV)

                    @pl.when((p // 4) == b)
                    def _(p=p, d=d):
                        sbuf[d - 1] = acc_ref[pl.ds(p * CH, CH), :].astype(
                            jnp.bfloat16)
                        rdma = pltpu.make_async_remote_copy(
                            src_ref=sbuf.at[d - 1], dst_ref=cbuf.at[d - 1],
                            send_sem=send_sems.at[d - 1],
                            recv_sem=recv_sems.at[d - 1],
                            device_id=(p,), device_id_type=_MESH)
                        rdma.start()

        if not _ALLREDUCE:
            return

        with jax.named_scope("rs_reduce"):
            my_rows = pl.ds(my * CH, CH)
            for e in range(1, N_DEV):
                wait_recv(e - 1)
                acc_ref[my_rows, :] += cbuf[e - 1].astype(jnp.float32)

        with jax.named_scope("ag"):
            sbuf[7] = acc_ref[my_rows, :].astype(jnp.bfloat16)
            for d in range(1, N_DEV):
                p = lax.rem(my + d, N_DEV)
                rdma = pltpu.make_async_remote_copy(
                    src_ref=sbuf.at[7], dst_ref=cbuf.at[6 + d],
                    send_sem=send_sems.at[6 + d],
                    recv_sem=recv_sems.at[6 + d],
                    device_id=(p,), device_id_type=_MESH)
                rdma.start()
                pending.append(rdma)
            for e in range(1, N_DEV):
                wait_recv(6 + e)
                c = lax.rem(my + (N_DEV - e), N_DEV)
                acc_ref[pl.ds(c * CH, CH), :] = cbuf[6 + e].astype(
                    jnp.float32)

        with jax.named_scope("store"):
            for b in range(B):
                out_ref[b] = acc_ref[b * SQ:(b + 1) * SQ, :]
            for j in range(N_DEV - 1):
                pltpu.make_async_remote_copy(
                    src_ref=sbuf.at[j], dst_ref=cbuf.at[j],
                    send_sem=send_sems.at[j], recv_sem=recv_sems.at[j],
                    device_id=(my,), device_id_type=_MESH).wait_send()
            for rdma in pending:
                rdma.wait_send()

    return pl.pallas_call(
        body,
        out_shape=jax.ShapeDtypeStruct((B, SQ, D), jnp.float32),
        in_specs=[
            pl.BlockSpec(memory_space=pltpu.MemorySpace.VMEM),
            pl.BlockSpec(memory_space=pltpu.MemorySpace.VMEM),
            pl.BlockSpec(memory_space=pltpu.MemorySpace.VMEM),
            pl.BlockSpec(memory_space=pltpu.MemorySpace.VMEM),
            pl.BlockSpec(memory_space=pltpu.MemorySpace.VMEM),
        ],
        out_specs=pl.BlockSpec(memory_space=pltpu.MemorySpace.VMEM),
        scratch_shapes=[
            pltpu.VMEM((ROWS, DM), jnp.bfloat16),
            pltpu.VMEM((ROWS, D), jnp.float32),
            pltpu.VMEM((8, CH, D), jnp.bfloat16),
            pltpu.VMEM((14, CH, D), jnp.bfloat16),
            pltpu.SemaphoreType.DMA((14,)),
            pltpu.SemaphoreType.DMA((14,)),
        ],
    )(x, Wq, Wo, Kmy, Vmy)


# device time: 34845 ns/iter; 1.0453x vs baseline; 1.0453x over previous
import jax
import jax.numpy as jnp
from jax import lax
from jax.experimental import pallas as pl
from jax.experimental.pallas import tpu as pltpu

N_DEV = 8
B, SQ, SKV, D = 2, 256, 512, 768
HQ_PER = 8
DH = 64
DM = HQ_PER * DH
SCALE = 0.125
ROWS = B * SQ
CH = ROWS // N_DEV

_MESH = pl.DeviceIdType.MESH
_ALLREDUCE = True


def kernel(x, Wq, Wo, K_ext, V_ext):
    my_sm = lax.axis_index("i")
    h0 = my_sm * HQ_PER
    Kmy = lax.dynamic_slice_in_dim(K_ext, h0, HQ_PER, axis=2)
    Vmy = lax.dynamic_slice_in_dim(V_ext, h0, HQ_PER, axis=2)
    Kmy = Kmy.astype(jnp.bfloat16).reshape(B, SKV, DM)
    Vmy = Vmy.astype(jnp.bfloat16).reshape(B, SKV, DM)

    def body(x_ref, wq_ref, wo_ref, k_ref, v_ref, out_ref,
             attn_ref, acc_ref, sbuf, cbuf, send_sems, recv_sems):
        my = lax.axis_index("i")
        r = my & 3
        myx = (r ^ (r >> 1)) & 1
        myy = r >> 1
        myz = my >> 2
        p_x = my ^ 1
        p_y = my ^ 3
        p_z = my ^ 4

        with jax.named_scope("qgemm"):
            x2 = x_ref[...].reshape(ROWS, D).astype(jnp.bfloat16)
            q_all = lax.dot_general(
                x2, wq_ref[...].astype(jnp.bfloat16),
                (((1,), (0,)), ((), ())),
                preferred_element_type=jnp.float32).astype(jnp.bfloat16)

        pending = []

        def wait_recv(j):
            rdma = pltpu.make_async_remote_copy(
                src_ref=sbuf.at[0], dst_ref=cbuf.at[j],
                send_sem=send_sems.at[j], recv_sem=recv_sems.at[j],
                device_id=(my,), device_id_type=_MESH)
            rdma.wait_recv()

        with jax.named_scope("attn"):
            for b in range(B):
                kb = k_ref[b]
                vb = v_ref[b]
                for h in range(HQ_PER):
                    q_bh = q_all[b * SQ:(b + 1) * SQ, h * DH:(h + 1) * DH]
                    k_bh = kb[:, h * DH:(h + 1) * DH]
                    v_bh = vb[:, h * DH:(h + 1) * DH]
                    s = lax.dot_general(
                        q_bh, k_bh, (((1,), (1,)), ((), ())),
                        preferred_element_type=jnp.float32) * SCALE
                    m = jnp.max(s, axis=1, keepdims=True)
                    p = jnp.exp(s - m)
                    l = jnp.sum(p, axis=1, keepdims=True)
                    o = lax.dot_general(
                        p.astype(jnp.bfloat16), v_bh, (((1,), (0,)), ((), ())),
                        preferred_element_type=jnp.float32)
                    attn_ref[pl.ds(b * SQ, SQ), pl.ds(h * DH, DH)] = (
                        o / l).astype(jnp.bfloat16)

        with jax.named_scope("ogemm"):
            acc_ref[...] = lax.dot_general(
                attn_ref[...], wo_ref[...].astype(jnp.bfloat16),
                (((1,), (0,)), ((), ())),
                preferred_element_type=jnp.float32)

        if not _ALLREDUCE:
            for b in range(B):
                out_ref[b] = acc_ref[b * SQ:(b + 1) * SQ, :]
            return

        with jax.named_scope("rs"):
            for d in range(1, N_DEV):
                p = lax.rem(my + d, N_DEV)
                sbuf[d - 1] = acc_ref[pl.ds(p * CH, CH), :].astype(
                    jnp.bfloat16)
                rdma = pltpu.make_async_remote_copy(
                    src_ref=sbuf.at[d - 1], dst_ref=cbuf.at[d - 1],
                    send_sem=send_sems.at[d - 1],
                    recv_sem=recv_sems.at[d - 1],
                    device_id=(p,), device_id_type=_MESH)
                rdma.start()
                pending.append(rdma)
            my_rows = pl.ds(my * CH, CH)
            for e in range(1, N_DEV):
                wait_recv(e - 1)
                acc_ref[my_rows, :] += cbuf[e - 1].astype(jnp.float32)

        with jax.named_scope("ag"):
            sbuf[7] = acc_ref[my_rows, :].astype(jnp.bfloat16)
            for d in range(1, N_DEV):
                p = lax.rem(my + d, N_DEV)
                rdma = pltpu.make_async_remote_copy(
                    src_ref=sbuf.at[7], dst_ref=cbuf.at[6 + d],
                    send_sem=send_sems.at[6 + d],
                    recv_sem=recv_sems.at[6 + d],
                    device_id=(p,), device_id_type=_MESH)
                rdma.start()
                pending.append(rdma)
            for e in range(1, N_DEV):
                wait_recv(6 + e)
                c = lax.rem(my + (N_DEV - e), N_DEV)
                acc_ref[pl.ds(c * CH, CH), :] = cbuf[6 + e].astype(
                    jnp.float32)

        with jax.named_scope("store"):
            for b in range(B):
                out_ref[b] = acc_ref[b * SQ:(b + 1) * SQ, :]
            for rdma in pending:
                rdma.wait_send()

    return pl.pallas_call(
        body,
        out_shape=jax.ShapeDtypeStruct((B, SQ, D), jnp.float32),
        in_specs=[
            pl.BlockSpec(memory_space=pltpu.MemorySpace.VMEM),
            pl.BlockSpec(memory_space=pltpu.MemorySpace.VMEM),
            pl.BlockSpec(memory_space=pltpu.MemorySpace.VMEM),
            pl.BlockSpec(memory_space=pltpu.MemorySpace.VMEM),
            pl.BlockSpec(memory_space=pltpu.MemorySpace.VMEM),
        ],
        out_specs=pl.BlockSpec(memory_space=pltpu.MemorySpace.VMEM),
        scratch_shapes=[
            pltpu.VMEM((ROWS, DM), jnp.bfloat16),
            pltpu.VMEM((ROWS, D), jnp.float32),
            pltpu.VMEM((8, CH, D), jnp.bfloat16),
            pltpu.VMEM((14, CH, D), jnp.bfloat16),
            pltpu.SemaphoreType.DMA((14,)),
            pltpu.SemaphoreType.DMA((14,)),
        ],
    )(x, Wq, Wo, Kmy, Vmy)
